# baseline (speedup 1.0000x reference)
import functools

import jax
import jax.numpy as jnp
from jax import lax
from jax.experimental import pallas as pl
from jax.experimental.pallas import tpu as pltpu
from jax.experimental.pallas import tpu_sc as plsc

SEQ_LEN = 4096
BATCH = 4
HIDDEN = 1024
LANES = 16
NUM_CORES = 2
NUM_SUBCORES = 16
NUM_WORKERS = NUM_CORES * NUM_SUBCORES
ROWS_TOTAL = SEQ_LEN * BATCH
ROWS_PER_WORKER = ROWS_TOTAL // NUM_WORKERS
GROUP = 32
NGROUPS = ROWS_PER_WORKER // GROUP
NSEM = 2
SEQ_PER_GROUP = GROUP // BATCH

_mesh = plsc.VectorSubcoreMesh(
    core_axis_name="c", subcore_axis_name="s",
    num_cores=NUM_CORES, num_subcores=NUM_SUBCORES,
)


@functools.partial(
    pl.kernel,
    out_type=jax.ShapeDtypeStruct((SEQ_LEN, BATCH, HIDDEN), jnp.float32),
    mesh=_mesh,
    scratch_types=[
        pltpu.VMEM((ROWS_PER_WORKER,), jnp.int32),
        pltpu.VMEM((2, HIDDEN), jnp.float32),
        pltpu.VMEM((2, 2, HIDDEN), jnp.float32),
        pltpu.VMEM((SEQ_PER_GROUP, BATCH, HIDDEN), jnp.float32),
        pltpu.SemaphoreType.DMA,
        pltpu.SemaphoreType.DMA,
        pltpu.SemaphoreType.DMA,
        pltpu.SemaphoreType.DMA,
    ],
)
def _sc_lookup(idx_hbm, table_hbm, out_hbm, idx_v, table_v, rep_v, dummy_v,
               ws0, ws1, ws2, ws3):
    wid = lax.axis_index("s") * NUM_CORES + lax.axis_index("c")
    pltpu.sync_copy(idx_hbm.at[pl.ds(wid * ROWS_PER_WORKER, ROWS_PER_WORKER)],
                    idx_v)
    pltpu.sync_copy(table_hbm, table_v)
    pltpu.sync_copy(table_hbm.at[0], rep_v.at[0, 0])
    pltpu.sync_copy(table_hbm.at[0], rep_v.at[0, 1])
    pltpu.sync_copy(table_hbm.at[1], rep_v.at[1, 0])
    pltpu.sync_copy(table_hbm.at[1], rep_v.at[1, 1])

    wsems = (ws0, ws1, ws2, ws3)
    seq0 = wid * (ROWS_PER_WORKER // BATCH)

    def drain_group(sem):
        pltpu.make_async_copy(
            dummy_v, out_hbm.at[pl.ds(seq0, SEQ_PER_GROUP)], sem).wait()

    def fire_group(g, sem):
        for h in range(GROUP // LANES):
            ids_vec = idx_v[pl.ds(g * GROUP + h * LANES, LANES)]
            for r in range(0, LANES, 2):
                flat = h * LANES + r
                i0 = ids_vec[r]
                i1 = ids_vec[r + 1]
                seq = seq0 + g * SEQ_PER_GROUP + flat // BATCH
                b = flat % BATCH
                eq = i0 == i1

                @pl.when(eq)
                def _():
                    pltpu.async_copy(
                        rep_v.at[i0], out_hbm.at[seq, pl.ds(b, 2)], sem)

                @pl.when(jnp.logical_not(eq))
                def _():
                    pltpu.async_copy(
                        table_v.at[i0], out_hbm.at[seq, b], sem)
                    pltpu.async_copy(
                        table_v.at[i1], out_hbm.at[seq, b + 1], sem)

    @pl.loop(0, NGROUPS, step=NSEM)
    def _(g0):
        for p in range(NSEM):
            g = g0 + p

            @pl.when(g >= NSEM)
            def _():
                drain_group(wsems[p])

            fire_group(g, wsems[p])

    for p in range(NSEM):
        drain_group(wsems[p])


def kernel(token_type_ids, segment_embedding_weight):
    ids = token_type_ids.reshape(ROWS_TOTAL).astype(jnp.int32)
    return _sc_lookup(ids, segment_embedding_weight)


# speedup vs baseline: 1.1650x; 1.1650x over previous
import functools

import jax
import jax.numpy as jnp
from jax import lax
from jax.experimental import pallas as pl
from jax.experimental.pallas import tpu as pltpu
from jax.experimental.pallas import tpu_sc as plsc

SEQ_LEN = 4096
BATCH = 4
HIDDEN = 1024
LANES = 16
NUM_CORES = 2
NUM_SUBCORES = 16
NUM_WORKERS = NUM_CORES * NUM_SUBCORES
ROWS_TOTAL = SEQ_LEN * BATCH
ROWS_PER_WORKER = ROWS_TOTAL // NUM_WORKERS
GROUP = 32
NGROUPS = ROWS_PER_WORKER // GROUP
NSEM = 2
SEQ_PER_GROUP = GROUP // BATCH

_mesh = plsc.VectorSubcoreMesh(
    core_axis_name="c", subcore_axis_name="s",
    num_cores=NUM_CORES, num_subcores=NUM_SUBCORES,
)


@functools.partial(
    pl.kernel,
    out_type=jax.ShapeDtypeStruct((SEQ_LEN, BATCH, HIDDEN), jnp.float32),
    mesh=_mesh,
    scratch_types=[
        pltpu.VMEM((ROWS_PER_WORKER,), jnp.int32),
        pltpu.VMEM((2, HIDDEN), jnp.float32),
        pltpu.VMEM((SEQ_PER_GROUP, BATCH, HIDDEN), jnp.float32),
        pltpu.SemaphoreType.DMA,
        pltpu.SemaphoreType.DMA,
    ],
)
def _sc_lookup(idx_hbm, table_hbm, out_hbm, idx_v, table_v, dummy_v,
               ws0, ws1):
    wid = lax.axis_index("s") * NUM_CORES + lax.axis_index("c")
    pltpu.sync_copy(idx_hbm.at[pl.ds(wid * ROWS_PER_WORKER, ROWS_PER_WORKER)],
                    idx_v)
    pltpu.sync_copy(table_hbm, table_v)

    wsems = (ws0, ws1)
    seq0 = wid * (ROWS_PER_WORKER // BATCH)

    def drain_group(sem):
        pltpu.make_async_copy(
            dummy_v, out_hbm.at[pl.ds(seq0, SEQ_PER_GROUP)], sem).wait()

    def fire_group(g, sem):
        for h in range(GROUP // LANES):
            ids_vec = idx_v[pl.ds(g * GROUP + h * LANES, LANES)]
            for r in range(LANES):
                flat = h * LANES + r
                src = table_v.at[ids_vec[r]]
                dst = out_hbm.at[seq0 + g * SEQ_PER_GROUP + flat // BATCH,
                                 flat % BATCH]
                pltpu.async_copy(src, dst, sem)

    @pl.loop(0, NGROUPS, step=NSEM)
    def _(g0):
        for p in range(NSEM):
            g = g0 + p

            @pl.when(g >= NSEM)
            def _():
                drain_group(wsems[p])

            fire_group(g, wsems[p])

    for p in range(NSEM):
        drain_group(wsems[p])


def kernel(token_type_ids, segment_embedding_weight):
    ids = token_type_ids.reshape(ROWS_TOTAL).astype(jnp.int32)
    return _sc_lookup(ids, segment_embedding_weight)
